# baseline (speedup 1.0000x reference)
import functools

import jax
import jax.numpy as jnp
from jax import lax
from jax.experimental import pallas as pl
from jax.experimental.pallas import tpu as pltpu
from jax.experimental.pallas import tpu_sc as plsc

D_MODEL = 64
SCALE = 8.0

_NC = 2
_NS = 16
_NW = _NC * _NS

_CHUNK = 128
_K = 2
_GROUP = _CHUNK * _K
_NBUF = 2


@functools.lru_cache(maxsize=None)
def _build(B):
    n_idx_rows = B // _CHUNK
    rows_per_w = n_idx_rows // _NW
    groups = rows_per_w // _K
    assert n_idx_rows % (_NW * _K) == 0 and groups % 2 == 0 and groups >= 4

    mesh = plsc.VectorSubcoreMesh(core_axis_name="c", subcore_axis_name="s")

    @functools.partial(
        pl.kernel,
        mesh=mesh,
        out_type=jax.ShapeDtypeStruct((B, 128), jnp.float32),
        compiler_params=pltpu.CompilerParams(use_tc_tiling_on_sc=False),
        scratch_types=[
            pltpu.VMEM((_NBUF, _K, _CHUNK), jnp.int32),
            pltpu.VMEM((_NBUF, _GROUP, D_MODEL), jnp.float32),
            pltpu.VMEM((_NBUF, _GROUP, D_MODEL), jnp.float32),
            pltpu.SemaphoreType.DMA,
            pltpu.SemaphoreType.DMA,
            pltpu.SemaphoreType.DMA,
            pltpu.SemaphoreType.DMA,
        ],
    )
    def emb(x_hbm, w_hbm, out_hbm, idx_v, rows_v, sbuf_v, g0, g1, s0, s1):
        wid = lax.axis_index("s") * _NC + lax.axis_index("c")
        gsem = (g0, g1)
        ssem = (s0, s1)
        base_row = wid * groups * _K

        def load_idx(g, b):
            pltpu.sync_copy(x_hbm.at[pl.ds(base_row + g * _K, _K)],
                            idx_v.at[b])
            for j in range(_K):
                for cc in range(_CHUNK // 16):
                    sl = pl.ds(cc * 16, 16)
                    idx_v[b, j, sl] = idx_v[b, j, sl] * 2

        def fire_gathers(b):
            for j in range(_K):
                pltpu.make_async_copy(
                    w_hbm.at[idx_v.at[b, j]],
                    rows_v.at[b, pl.ds(j * _CHUNK, _CHUNK)],
                    gsem[b],
                ).start()

        def drain_gathers(b):
            for j in range(_K):
                pltpu.make_async_copy(
                    w_hbm.at[idx_v.at[b, j]],
                    rows_v.at[b, pl.ds(j * _CHUNK, _CHUNK)],
                    gsem[b],
                ).wait()

        def scale(b):
            @plsc.parallel_loop(0, _GROUP, unroll=8)
            def _(r):
                for cc in range(D_MODEL // 16):
                    sl = pl.ds(cc * 16, 16)
                    sbuf_v[b, r, sl] = rows_v[b, r, sl] * SCALE

        def fire_store(g, b):
            pltpu.make_async_copy(
                sbuf_v.at[b],
                out_hbm.at[pl.ds((base_row + g * _K) * _CHUNK, _GROUP),
                           pl.ds(0, D_MODEL)],
                ssem[b],
            ).start()

        def drain_store(g, b):
            pltpu.make_async_copy(
                sbuf_v.at[b],
                out_hbm.at[pl.ds((base_row + g * _K) * _CHUNK, _GROUP),
                           pl.ds(0, D_MODEL)],
                ssem[b],
            ).wait()

        load_idx(0, 0)
        fire_gathers(0)
        load_idx(1, 1)
        fire_gathers(1)
        for b in range(_NBUF):
            drain_gathers(b)
            scale(b)
            fire_store(b, b)
            load_idx(b + 2, b)
            fire_gathers(b)

        def outer_body(outer, carry):
            for b in range(_NBUF):
                g = outer * _NBUF + b
                drain_gathers(b)
                drain_store(g - 2, b)
                scale(b)
                fire_store(g, b)
                load_idx(g + 2, b)
                fire_gathers(b)
            return carry

        lax.fori_loop(1, (groups - 2) // _NBUF, outer_body, 0)

        for b in range(_NBUF):
            g = groups - 2 + b
            drain_gathers(b)
            drain_store(g - 2, b)
            scale(b)
            fire_store(g, b)
        for b in range(_NBUF):
            drain_store(groups - 2 + b, b)

    return emb


def kernel(x, W):
    B = x.size
    x2d = x.reshape(B // _CHUNK, _CHUNK)
    W2 = jnp.pad(W, ((0, 0), (0, 128 - D_MODEL))).reshape(2 * W.shape[0],
                                                          D_MODEL)
    out = _build(B)(x2d, W2)
    return out[:, :D_MODEL].reshape(x.shape + (D_MODEL,))


# speedup vs baseline: 1.0095x; 1.0095x over previous
import functools

import jax
import jax.numpy as jnp
from jax import lax
from jax.experimental import pallas as pl
from jax.experimental.pallas import tpu as pltpu
from jax.experimental.pallas import tpu_sc as plsc

D_MODEL = 64
SCALE = 8.0

_NC = 2
_NS = 16
_NW = _NC * _NS

_CHUNK = 128
_K = 5
_GROUP = _CHUNK * _K
_NBUF = 2


@functools.lru_cache(maxsize=None)
def _build(B):
    n_idx_rows = B // _CHUNK
    rows_per_w = n_idx_rows // _NW
    groups = rows_per_w // _K
    assert n_idx_rows % (_NW * _K) == 0 and groups % 2 == 0 and groups >= 4

    mesh = plsc.VectorSubcoreMesh(core_axis_name="c", subcore_axis_name="s")

    @functools.partial(
        pl.kernel,
        mesh=mesh,
        out_type=jax.ShapeDtypeStruct((B, 128), jnp.float32),
        compiler_params=pltpu.CompilerParams(use_tc_tiling_on_sc=False),
        scratch_types=[
            pltpu.VMEM((_NBUF, _K, _CHUNK), jnp.int32),
            pltpu.VMEM((_NBUF, _GROUP, D_MODEL), jnp.float32),
            pltpu.SemaphoreType.DMA,
            pltpu.SemaphoreType.DMA,
            pltpu.SemaphoreType.DMA,
            pltpu.SemaphoreType.DMA,
        ],
    )
    def emb(x_hbm, w_hbm, out_hbm, idx_v, rows_v, g0, g1, s0, s1):
        wid = lax.axis_index("s") * _NC + lax.axis_index("c")
        gsem = (g0, g1)
        ssem = (s0, s1)
        base_row = wid * groups * _K

        def load_idx(g, b):
            pltpu.sync_copy(x_hbm.at[pl.ds(base_row + g * _K, _K)],
                            idx_v.at[b])
            for j in range(_K):
                for cc in range(_CHUNK // 16):
                    sl = pl.ds(cc * 16, 16)
                    idx_v[b, j, sl] = idx_v[b, j, sl] * 2

        def fire_gathers(b):
            for j in range(_K):
                pltpu.make_async_copy(
                    w_hbm.at[idx_v.at[b, j]],
                    rows_v.at[b, pl.ds(j * _CHUNK, _CHUNK)],
                    gsem[b],
                ).start()

        def drain_gathers(b):
            for j in range(_K):
                pltpu.make_async_copy(
                    w_hbm.at[idx_v.at[b, j]],
                    rows_v.at[b, pl.ds(j * _CHUNK, _CHUNK)],
                    gsem[b],
                ).wait()

        def scale(b):
            @plsc.parallel_loop(0, _GROUP, unroll=8)
            def _(r):
                for cc in range(D_MODEL // 16):
                    sl = pl.ds(cc * 16, 16)
                    rows_v[b, r, sl] = rows_v[b, r, sl] * SCALE

        def fire_store(g, b):
            pltpu.make_async_copy(
                rows_v.at[b],
                out_hbm.at[pl.ds((base_row + g * _K) * _CHUNK, _GROUP),
                           pl.ds(0, D_MODEL)],
                ssem[b],
            ).start()

        def drain_store(g, b):
            pltpu.make_async_copy(
                rows_v.at[b],
                out_hbm.at[pl.ds((base_row + g * _K) * _CHUNK, _GROUP),
                           pl.ds(0, D_MODEL)],
                ssem[b],
            ).wait()

        load_idx(0, 0)
        fire_gathers(0)
        load_idx(1, 1)
        fire_gathers(1)

        def outer_body(outer, carry):
            for b in range(_NBUF):
                g = outer * _NBUF + b
                drain_gathers(b)
                scale(b)
                fire_store(g, b)
                load_idx(g + 2, b)
                drain_store(g, b)
                fire_gathers(b)
            return carry

        lax.fori_loop(0, (groups - 2) // _NBUF, outer_body, 0)

        for b in range(_NBUF):
            g = groups - 2 + b
            drain_gathers(b)
            scale(b)
            fire_store(g, b)
        for b in range(_NBUF):
            drain_store(groups - 2 + b, b)

    return emb


def kernel(x, W):
    B = x.size
    x2d = x.reshape(B // _CHUNK, _CHUNK)
    W2 = jnp.pad(W, ((0, 0), (0, 128 - D_MODEL))).reshape(2 * W.shape[0],
                                                          D_MODEL)
    out = _build(B)(x2d, W2)
    return out[:, :D_MODEL].reshape(x.shape + (D_MODEL,))


# speedup vs baseline: 1.0104x; 1.0009x over previous
import functools

import jax
import jax.numpy as jnp
from jax import lax
from jax.experimental import pallas as pl
from jax.experimental.pallas import tpu as pltpu
from jax.experimental.pallas import tpu_sc as plsc

D_MODEL = 64
SCALE = 8.0

_NC = 2
_NS = 16
_NW = _NC * _NS

_CHUNK = 128
_K = 4
_GROUP = _CHUNK * _K
_NBUF = 2


@functools.lru_cache(maxsize=None)
def _build(B):
    n_idx_rows = B // _CHUNK
    rows_per_w = n_idx_rows // _NW
    groups = rows_per_w // _K
    assert n_idx_rows % (_NW * _K) == 0 and groups % 2 == 0 and groups >= 4

    mesh = plsc.VectorSubcoreMesh(core_axis_name="c", subcore_axis_name="s")

    @functools.partial(
        pl.kernel,
        mesh=mesh,
        out_type=jax.ShapeDtypeStruct((B, 128), jnp.float32),
        compiler_params=pltpu.CompilerParams(use_tc_tiling_on_sc=False),
        scratch_types=[
            pltpu.VMEM((_NBUF, _K, _CHUNK), jnp.int32),
            pltpu.VMEM((_NBUF, _GROUP, D_MODEL), jnp.float32),
            pltpu.SemaphoreType.DMA,
            pltpu.SemaphoreType.DMA,
            pltpu.SemaphoreType.DMA,
            pltpu.SemaphoreType.DMA,
        ],
    )
    def emb(x_hbm, w_hbm, out_hbm, idx_v, rows_v, g0, g1, s0, s1):
        wid = lax.axis_index("s") * _NC + lax.axis_index("c")
        gsem = (g0, g1)
        ssem = (s0, s1)
        base_row = wid * groups * _K

        def load_idx(g, b):
            pltpu.sync_copy(x_hbm.at[pl.ds(base_row + g * _K, _K)],
                            idx_v.at[b])
            for j in range(_K):
                for cc in range(_CHUNK // 16):
                    sl = pl.ds(cc * 16, 16)
                    idx_v[b, j, sl] = idx_v[b, j, sl] * 2

        def fire_gathers(b):
            for j in range(_K):
                pltpu.make_async_copy(
                    w_hbm.at[idx_v.at[b, j]],
                    rows_v.at[b, pl.ds(j * _CHUNK, _CHUNK)],
                    gsem[b],
                ).start()

        def drain_gathers(b):
            for j in range(_K):
                pltpu.make_async_copy(
                    w_hbm.at[idx_v.at[b, j]],
                    rows_v.at[b, pl.ds(j * _CHUNK, _CHUNK)],
                    gsem[b],
                ).wait()

        def scale(b):
            @plsc.parallel_loop(0, _GROUP, unroll=8)
            def _(r):
                for cc in range(D_MODEL // 16):
                    sl = pl.ds(cc * 16, 16)
                    rows_v[b, r, sl] = rows_v[b, r, sl] * SCALE

        def fire_store(g, b):
            pltpu.make_async_copy(
                rows_v.at[b],
                out_hbm.at[pl.ds((base_row + g * _K) * _CHUNK, _GROUP),
                           pl.ds(0, D_MODEL)],
                ssem[b],
            ).start()

        def drain_store(g, b):
            pltpu.make_async_copy(
                rows_v.at[b],
                out_hbm.at[pl.ds((base_row + g * _K) * _CHUNK, _GROUP),
                           pl.ds(0, D_MODEL)],
                ssem[b],
            ).wait()

        load_idx(0, 0)
        fire_gathers(0)
        load_idx(1, 1)
        fire_gathers(1)

        def outer_body(outer, carry):
            for b in range(_NBUF):
                g = outer * _NBUF + b
                drain_gathers(b)
                scale(b)
                fire_store(g, b)
                load_idx(g + 2, b)
                drain_store(g, b)
                fire_gathers(b)
            return carry

        lax.fori_loop(0, (groups - 2) // _NBUF, outer_body, 0)

        for b in range(_NBUF):
            g = groups - 2 + b
            drain_gathers(b)
            scale(b)
            fire_store(g, b)
        for b in range(_NBUF):
            drain_store(groups - 2 + b, b)

    return emb


def kernel(x, W):
    B = x.size
    x2d = x.reshape(B // _CHUNK, _CHUNK)
    W2 = jnp.pad(W, ((0, 0), (0, 128 - D_MODEL))).reshape(2 * W.shape[0],
                                                          D_MODEL)
    out = _build(B)(x2d, W2)
    return out[:, :D_MODEL].reshape(x.shape + (D_MODEL,))
